# baseline (device time: 442315 ns/iter reference)
import numpy as np

import jax
import jax.numpy as jnp
from jax import lax
from jax.experimental import pallas as pl
from jax.experimental.pallas import tpu as pltpu

N_DEV = 8
SQ = 2048
D = 1024
DH = 128
H_LOC = D // DH
QB = 512
CHUNK = SQ // N_DEV
SCALE = 0.08838834764831843

_inv = 1.0 / (10000.0 ** (np.arange(0, DH, 2) / DH))
_pos = np.arange(SQ)[:, None] * _inv[None, :]
_COS = jnp.asarray(np.repeat(np.cos(_pos), 2, axis=-1), dtype=jnp.float32)
_SIN = jnp.asarray(np.repeat(np.sin(_pos), 2, axis=-1), dtype=jnp.float32)


def _qkv_body(x_ref, wq_ref, wk_ref, wv_ref, q_ref, k_ref, v_ref):
    x = x_ref[...].astype(jnp.bfloat16)
    q_ref[...] = jnp.dot(x, wq_ref[...].astype(jnp.bfloat16),
                         preferred_element_type=jnp.float32)
    k_ref[...] = jnp.dot(x, wk_ref[...].astype(jnp.bfloat16),
                         preferred_element_type=jnp.float32)
    v_ref[...] = jnp.dot(x, wv_ref[...].astype(jnp.bfloat16),
                         preferred_element_type=jnp.float32)


def _qkv(x2, Wq, Wk, Wv):
    w_spec = pl.BlockSpec((D, DH), lambda h: (0, h))
    o_spec = pl.BlockSpec((SQ, DH), lambda h: (0, h))
    o_shape = jax.ShapeDtypeStruct((SQ, D), jnp.float32)
    return pl.pallas_call(
        _qkv_body,
        grid=(H_LOC,),
        in_specs=[pl.BlockSpec((SQ, D), lambda h: (0, 0)), w_spec, w_spec, w_spec],
        out_specs=[o_spec, o_spec, o_spec],
        out_shape=[o_shape, o_shape, o_shape],
    )(x2, Wq, Wk, Wv)


def _rope(t):
    th = t.reshape(SQ, H_LOC, DH // 2, 2)
    tr = jnp.stack([-th[..., 1], th[..., 0]], axis=-1).reshape(SQ, H_LOC, DH)
    return (
        t.reshape(SQ, H_LOC, DH) * _COS[:, None, :] + tr * _SIN[:, None, :]
    ).reshape(SQ, D)


def _attn_body(q_ref, k_ref, v_ref, o_ref):
    s = lax.dot_general(
        q_ref[...].astype(jnp.bfloat16), k_ref[...].astype(jnp.bfloat16),
        (((1,), (1,)), ((), ())),
        preferred_element_type=jnp.float32,
    ) * SCALE
    m = jnp.max(s, axis=-1, keepdims=True)
    w = jnp.exp(s - m)
    w = w / jnp.sum(w, axis=-1, keepdims=True)
    o_ref[...] = jnp.dot(w.astype(jnp.bfloat16),
                         v_ref[...].astype(jnp.bfloat16),
                         preferred_element_type=jnp.float32)


def _attention(q, k, v):
    kv_spec = pl.BlockSpec((SQ, DH), lambda h, qb: (0, h))
    return pl.pallas_call(
        _attn_body,
        grid=(H_LOC, SQ // QB),
        in_specs=[
            pl.BlockSpec((QB, DH), lambda h, qb: (qb, h)),
            kv_spec,
            kv_spec,
        ],
        out_specs=pl.BlockSpec((QB, DH), lambda h, qb: (qb, h)),
        out_shape=jax.ShapeDtypeStruct((SQ, D), jnp.float32),
    )(q, k, v)


def _proj_ar_body(ctx_ref, wo_ref, o_ref, comm_ref, send_sems, recv_sems):
    me = lax.axis_index("i")
    left = lax.rem(me + N_DEV - 1, N_DEV)
    right = lax.rem(me + 1, N_DEV)

    wo = wo_ref[...].astype(jnp.bfloat16)
    for c in range(N_DEV):
        o_ref[c] = jnp.dot(
            ctx_ref[pl.ds(c * CHUNK, CHUNK), :].astype(jnp.bfloat16), wo,
            preferred_element_type=jnp.float32,
        )

    barrier = pltpu.get_barrier_semaphore()
    for nbr in (left, right):
        pl.semaphore_signal(
            barrier, inc=1, device_id=(nbr,),
            device_id_type=pl.DeviceIdType.MESH,
        )
    pl.semaphore_wait(barrier, 2)

    for h in range(N_DEV - 1):
        send_idx = lax.rem(me - h + N_DEV, N_DEV)
        recv_idx = lax.rem(me - h - 1 + N_DEV, N_DEV)
        rdma = pltpu.make_async_remote_copy(
            src_ref=o_ref.at[send_idx],
            dst_ref=comm_ref.at[h],
            send_sem=send_sems.at[h],
            recv_sem=recv_sems.at[h],
            device_id=(right,),
            device_id_type=pl.DeviceIdType.MESH,
        )
        rdma.start()
        rdma.wait()
        o_ref[recv_idx] = o_ref[recv_idx] + comm_ref[h]

    for h in range(N_DEV - 1):
        send_idx = lax.rem(me + 1 - h + N_DEV, N_DEV)
        rdma = pltpu.make_async_remote_copy(
            src_ref=o_ref.at[send_idx],
            dst_ref=o_ref.at[send_idx],
            send_sem=send_sems.at[N_DEV - 1 + h],
            recv_sem=recv_sems.at[N_DEV - 1 + h],
            device_id=(right,),
            device_id_type=pl.DeviceIdType.MESH,
        )
        rdma.start()
        rdma.wait()


def _proj_allreduce(ctx, Wo):
    n_sems = 2 * (N_DEV - 1)
    return pl.pallas_call(
        _proj_ar_body,
        in_specs=[
            pl.BlockSpec(memory_space=pltpu.VMEM),
            pl.BlockSpec(memory_space=pltpu.VMEM),
        ],
        out_specs=pl.BlockSpec(memory_space=pltpu.VMEM),
        out_shape=jax.ShapeDtypeStruct((N_DEV, CHUNK, D), jnp.float32),
        scratch_shapes=[
            pltpu.VMEM((N_DEV - 1, CHUNK, D), jnp.float32),
            pltpu.SemaphoreType.DMA((n_sems,)),
            pltpu.SemaphoreType.DMA((n_sems,)),
        ],
        compiler_params=pltpu.CompilerParams(collective_id=0),
    )(ctx, Wo)


def kernel(x, Wq, Wk, Wv, Wo):
    x2 = x.reshape(SQ, D)
    q, k, v = _qkv(x2, Wq, Wk, Wv)
    q = _rope(q)
    k = _rope(k)
    ctx = _attention(q, k, v)
    out = _proj_allreduce(ctx, Wo)
    return out.reshape(1, SQ, D)


# device time: 352810 ns/iter; 1.2537x vs baseline; 1.2537x over previous
import numpy as np

import jax
import jax.numpy as jnp
from jax import lax
from jax.experimental import pallas as pl
from jax.experimental.pallas import tpu as pltpu

N_DEV = 8
SQ = 2048
D = 1024
DH = 128
H_LOC = D // DH
QB = 512
CHUNK = SQ // N_DEV
SCALE = 0.08838834764831843

_inv = 1.0 / (10000.0 ** (np.arange(0, DH, 2) / DH))
_pos = np.arange(SQ)[:, None] * _inv[None, :]
_COS = jnp.asarray(np.repeat(np.cos(_pos), 2, axis=-1), dtype=jnp.float32)
_SIN = jnp.asarray(np.repeat(np.sin(_pos), 2, axis=-1), dtype=jnp.float32)

_Rnp = np.zeros((DH, DH), dtype=np.float32)
for _k in range(DH // 2):
    _Rnp[2 * _k + 1, 2 * _k] = -1.0
    _Rnp[2 * _k, 2 * _k + 1] = 1.0
_ROT = jnp.asarray(_Rnp)


def _qkv_body(x_ref, wq_ref, wk_ref, wv_ref, cos_ref, sin_ref, rot_ref,
              q_ref, k_ref, v_ref):
    x = x_ref[...].astype(jnp.bfloat16)
    rot = rot_ref[...].astype(jnp.bfloat16)
    cos = cos_ref[...]
    sin = sin_ref[...]

    def rope(t):
        tr = jnp.dot(t.astype(jnp.bfloat16), rot,
                     preferred_element_type=jnp.float32)
        return t * cos + tr * sin

    q = jnp.dot(x, wq_ref[...].astype(jnp.bfloat16),
                preferred_element_type=jnp.float32)
    k = jnp.dot(x, wk_ref[...].astype(jnp.bfloat16),
                preferred_element_type=jnp.float32)
    q_ref[...] = rope(q)
    k_ref[...] = rope(k)
    v_ref[...] = jnp.dot(x, wv_ref[...].astype(jnp.bfloat16),
                         preferred_element_type=jnp.float32)


def _qkv(x2, Wq, Wk, Wv):
    w_spec = pl.BlockSpec((D, DH), lambda h: (0, h))
    t_spec = pl.BlockSpec((SQ, DH), lambda h: (0, 0))
    o_spec = pl.BlockSpec((SQ, DH), lambda h: (0, h))
    o_shape = jax.ShapeDtypeStruct((SQ, D), jnp.float32)
    return pl.pallas_call(
        _qkv_body,
        grid=(H_LOC,),
        in_specs=[
            pl.BlockSpec((SQ, D), lambda h: (0, 0)),
            w_spec, w_spec, w_spec,
            t_spec, t_spec,
            pl.BlockSpec((DH, DH), lambda h: (0, 0)),
        ],
        out_specs=[o_spec, o_spec, o_spec],
        out_shape=[o_shape, o_shape, o_shape],
    )(x2, Wq, Wk, Wv, _COS, _SIN, _ROT)


def _attn_body(q_ref, k_ref, v_ref, o_ref):
    s = lax.dot_general(
        q_ref[...].astype(jnp.bfloat16), k_ref[...].astype(jnp.bfloat16),
        (((1,), (1,)), ((), ())),
        preferred_element_type=jnp.float32,
    ) * SCALE
    m = jnp.max(s, axis=-1, keepdims=True)
    w = jnp.exp(s - m)
    w = w / jnp.sum(w, axis=-1, keepdims=True)
    o_ref[...] = jnp.dot(w.astype(jnp.bfloat16),
                         v_ref[...].astype(jnp.bfloat16),
                         preferred_element_type=jnp.float32)


def _attention(q, k, v):
    kv_spec = pl.BlockSpec((SQ, DH), lambda h, qb: (0, h))
    return pl.pallas_call(
        _attn_body,
        grid=(H_LOC, SQ // QB),
        in_specs=[
            pl.BlockSpec((QB, DH), lambda h, qb: (qb, h)),
            kv_spec,
            kv_spec,
        ],
        out_specs=pl.BlockSpec((QB, DH), lambda h, qb: (qb, h)),
        out_shape=jax.ShapeDtypeStruct((SQ, D), jnp.float32),
    )(q, k, v)


def _proj_ar_body(ctx_ref, wo_ref, o_ref, comm_ref, send_sems, recv_sems):
    me = lax.axis_index("i")
    left = lax.rem(me + N_DEV - 1, N_DEV)
    right = lax.rem(me + 1, N_DEV)

    wo = wo_ref[...].astype(jnp.bfloat16)
    for c in range(N_DEV):
        o_ref[c] = jnp.dot(
            ctx_ref[pl.ds(c * CHUNK, CHUNK), :].astype(jnp.bfloat16), wo,
            preferred_element_type=jnp.float32,
        )

    barrier = pltpu.get_barrier_semaphore()
    for nbr in (left, right):
        pl.semaphore_signal(
            barrier, inc=1, device_id=(nbr,),
            device_id_type=pl.DeviceIdType.MESH,
        )
    pl.semaphore_wait(barrier, 2)

    for h in range(N_DEV - 1):
        send_idx = lax.rem(me - h + N_DEV, N_DEV)
        recv_idx = lax.rem(me - h - 1 + N_DEV, N_DEV)
        rdma = pltpu.make_async_remote_copy(
            src_ref=o_ref.at[send_idx],
            dst_ref=comm_ref.at[h],
            send_sem=send_sems.at[h],
            recv_sem=recv_sems.at[h],
            device_id=(right,),
            device_id_type=pl.DeviceIdType.MESH,
        )
        rdma.start()
        rdma.wait()
        o_ref[recv_idx] = o_ref[recv_idx] + comm_ref[h]

    for h in range(N_DEV - 1):
        send_idx = lax.rem(me + 1 - h + N_DEV, N_DEV)
        rdma = pltpu.make_async_remote_copy(
            src_ref=o_ref.at[send_idx],
            dst_ref=o_ref.at[send_idx],
            send_sem=send_sems.at[N_DEV - 1 + h],
            recv_sem=recv_sems.at[N_DEV - 1 + h],
            device_id=(right,),
            device_id_type=pl.DeviceIdType.MESH,
        )
        rdma.start()
        rdma.wait()


def _proj_allreduce(ctx, Wo):
    n_sems = 2 * (N_DEV - 1)
    return pl.pallas_call(
        _proj_ar_body,
        in_specs=[
            pl.BlockSpec(memory_space=pltpu.VMEM),
            pl.BlockSpec(memory_space=pltpu.VMEM),
        ],
        out_specs=pl.BlockSpec(memory_space=pltpu.VMEM),
        out_shape=jax.ShapeDtypeStruct((N_DEV, CHUNK, D), jnp.float32),
        scratch_shapes=[
            pltpu.VMEM((N_DEV - 1, CHUNK, D), jnp.float32),
            pltpu.SemaphoreType.DMA((n_sems,)),
            pltpu.SemaphoreType.DMA((n_sems,)),
        ],
        compiler_params=pltpu.CompilerParams(collective_id=0),
    )(ctx, Wo)


def kernel(x, Wq, Wk, Wv, Wo):
    x2 = x.reshape(SQ, D)
    q, k, v = _qkv(x2, Wq, Wk, Wv)
    ctx = _attention(q, k, v)
    out = _proj_allreduce(ctx, Wo)
    return out.reshape(1, SQ, D)


# device time: 261786 ns/iter; 1.6896x vs baseline; 1.3477x over previous
import numpy as np

import jax
import jax.numpy as jnp
from jax import lax
from jax.experimental import pallas as pl
from jax.experimental.pallas import tpu as pltpu

N_DEV = 8
SQ = 2048
D = 1024
DH = 128
H_LOC = D // DH
QB = 512
CHUNK = SQ // N_DEV
SCALE = 0.08838834764831843

_inv = 1.0 / (10000.0 ** (np.arange(0, DH, 2) / DH))
_pos = np.arange(SQ)[:, None] * _inv[None, :]
_COS = jnp.asarray(np.repeat(np.cos(_pos), 2, axis=-1), dtype=jnp.float32)
_SIN = jnp.asarray(np.repeat(np.sin(_pos), 2, axis=-1), dtype=jnp.float32)

_Rnp = np.zeros((DH, DH), dtype=np.float32)
for _k in range(DH // 2):
    _Rnp[2 * _k + 1, 2 * _k] = -1.0
    _Rnp[2 * _k, 2 * _k + 1] = 1.0
_ROT = jnp.asarray(_Rnp)


def _qkv_body(x_ref, wq_ref, wk_ref, wv_ref, cos_ref, sin_ref, rot_ref,
              q_ref, k_ref, v_ref):
    x = x_ref[...].astype(jnp.bfloat16)
    rot = rot_ref[...].astype(jnp.bfloat16)
    cos = cos_ref[...]
    sin = sin_ref[...]

    def rope(t):
        tr = jnp.dot(t.astype(jnp.bfloat16), rot,
                     preferred_element_type=jnp.float32)
        return t * cos + tr * sin

    q = jnp.dot(x, wq_ref[...].astype(jnp.bfloat16),
                preferred_element_type=jnp.float32)
    k = jnp.dot(x, wk_ref[...].astype(jnp.bfloat16),
                preferred_element_type=jnp.float32)
    q_ref[...] = rope(q)
    k_ref[...] = rope(k)
    v_ref[...] = jnp.dot(x, wv_ref[...].astype(jnp.bfloat16),
                         preferred_element_type=jnp.float32)


def _qkv(x2, Wq, Wk, Wv):
    w_spec = pl.BlockSpec((D, DH), lambda h: (0, h))
    t_spec = pl.BlockSpec((SQ, DH), lambda h: (0, 0))
    o_spec = pl.BlockSpec((SQ, DH), lambda h: (0, h))
    o_shape = jax.ShapeDtypeStruct((SQ, D), jnp.float32)
    return pl.pallas_call(
        _qkv_body,
        grid=(H_LOC,),
        in_specs=[
            pl.BlockSpec((SQ, D), lambda h: (0, 0)),
            w_spec, w_spec, w_spec,
            t_spec, t_spec,
            pl.BlockSpec((DH, DH), lambda h: (0, 0)),
        ],
        out_specs=[o_spec, o_spec, o_spec],
        out_shape=[o_shape, o_shape, o_shape],
    )(x2, Wq, Wk, Wv, _COS, _SIN, _ROT)


def _attn_body(q_ref, k_ref, v_ref, o_ref):
    s = lax.dot_general(
        q_ref[...].astype(jnp.bfloat16), k_ref[...].astype(jnp.bfloat16),
        (((1,), (1,)), ((), ())),
        preferred_element_type=jnp.float32,
    ) * SCALE
    m = jnp.max(s, axis=-1, keepdims=True)
    w = jnp.exp(s - m)
    w = w / jnp.sum(w, axis=-1, keepdims=True)
    o_ref[...] = jnp.dot(w.astype(jnp.bfloat16),
                         v_ref[...].astype(jnp.bfloat16),
                         preferred_element_type=jnp.float32)


def _attention(q, k, v):
    kv_spec = pl.BlockSpec((SQ, DH), lambda h, qb: (0, h))
    return pl.pallas_call(
        _attn_body,
        grid=(H_LOC, SQ // QB),
        in_specs=[
            pl.BlockSpec((QB, DH), lambda h, qb: (qb, h)),
            kv_spec,
            kv_spec,
        ],
        out_specs=pl.BlockSpec((QB, DH), lambda h, qb: (qb, h)),
        out_shape=jax.ShapeDtypeStruct((SQ, D), jnp.float32),
    )(q, k, v)


_RS_BASE = (0, 4, 6)
_AG_BASE = {2: 7, 1: 8, 0: 10}


def _proj_ar_body(ctx_ref, wo_ref, o_ref, comm_ref, p16_ref,
                  send_sems, recv_sems):
    me = lax.axis_index("i")

    wo = wo_ref[...].astype(jnp.bfloat16)
    for c in range(N_DEV):
        p = jnp.dot(
            ctx_ref[pl.ds(c * CHUNK, CHUNK), :].astype(jnp.bfloat16), wo,
            preferred_element_type=jnp.float32,
        )
        o_ref[c] = p
        p16_ref[c] = p.astype(jnp.bfloat16)

    barrier = pltpu.get_barrier_semaphore()
    for s in range(3):
        pl.semaphore_signal(
            barrier, inc=1, device_id=(me ^ (1 << s),),
            device_id_type=pl.DeviceIdType.MESH,
        )
    pl.semaphore_wait(barrier, 3)

    for s in range(3):
        partner = me ^ (1 << s)
        low = me & ((1 << s) - 1)
        mebit = (me >> s) & 1
        pbit = 1 - mebit
        rdmas = []
        for j in range(N_DEV >> (s + 1)):
            c_send = j * (1 << (s + 1)) + pbit * (1 << s) + low
            slot = _RS_BASE[s] + j
            rdma = pltpu.make_async_remote_copy(
                src_ref=p16_ref.at[c_send],
                dst_ref=comm_ref.at[slot],
                send_sem=send_sems.at[slot],
                recv_sem=recv_sems.at[slot],
                device_id=(partner,),
                device_id_type=pl.DeviceIdType.MESH,
            )
            rdma.start()
            rdmas.append(rdma)
        for rdma in rdmas:
            rdma.wait()
        for j in range(N_DEV >> (s + 1)):
            c_keep = j * (1 << (s + 1)) + mebit * (1 << s) + low
            slot = _RS_BASE[s] + j
            acc = o_ref[c_keep] + comm_ref[slot].astype(jnp.float32)
            o_ref[c_keep] = acc
            p16_ref[c_keep] = acc.astype(jnp.bfloat16)

    for s in (2, 1, 0):
        partner = me ^ (1 << s)
        lowmask = (1 << (s + 1)) - 1
        mylow = me & lowmask
        plow = partner & lowmask
        rdmas = []
        for j in range(N_DEV >> (s + 1)):
            c_send = j * (1 << (s + 1)) + mylow
            slot = _AG_BASE[s] + j
            rdma = pltpu.make_async_remote_copy(
                src_ref=p16_ref.at[c_send],
                dst_ref=comm_ref.at[slot],
                send_sem=send_sems.at[slot],
                recv_sem=recv_sems.at[slot],
                device_id=(partner,),
                device_id_type=pl.DeviceIdType.MESH,
            )
            rdma.start()
            rdmas.append(rdma)
        for rdma in rdmas:
            rdma.wait()
        for j in range(N_DEV >> (s + 1)):
            c_recv = j * (1 << (s + 1)) + plow
            slot = _AG_BASE[s] + j
            o_ref[c_recv] = comm_ref[slot].astype(jnp.float32)
            p16_ref[c_recv] = comm_ref[slot]


def _proj_allreduce(ctx, Wo):
    n_sems = 2 * (N_DEV - 1)
    return pl.pallas_call(
        _proj_ar_body,
        in_specs=[
            pl.BlockSpec(memory_space=pltpu.VMEM),
            pl.BlockSpec(memory_space=pltpu.VMEM),
        ],
        out_specs=pl.BlockSpec(memory_space=pltpu.VMEM),
        out_shape=jax.ShapeDtypeStruct((N_DEV, CHUNK, D), jnp.float32),
        scratch_shapes=[
            pltpu.VMEM((2 * (N_DEV - 1), CHUNK, D), jnp.bfloat16),
            pltpu.VMEM((N_DEV, CHUNK, D), jnp.bfloat16),
            pltpu.SemaphoreType.DMA((n_sems,)),
            pltpu.SemaphoreType.DMA((n_sems,)),
        ],
        compiler_params=pltpu.CompilerParams(collective_id=0),
    )(ctx, Wo)


def kernel(x, Wq, Wk, Wv, Wo):
    x2 = x.reshape(SQ, D)
    q, k, v = _qkv(x2, Wq, Wk, Wv)
    ctx = _attention(q, k, v)
    out = _proj_allreduce(ctx, Wo)
    return out.reshape(1, SQ, D)


# device time: 205837 ns/iter; 2.1489x vs baseline; 1.2718x over previous
import numpy as np

import jax
import jax.numpy as jnp
from jax import lax
from jax.experimental import pallas as pl
from jax.experimental.pallas import tpu as pltpu

N_DEV = 8
SQ = 2048
D = 1024
DH = 128
H_LOC = D // DH
QB = 512
CHUNK = SQ // N_DEV
SCALE = 0.08838834764831843

_inv = 1.0 / (10000.0 ** (np.arange(0, DH, 2) / DH))
_pos = np.arange(SQ)[:, None] * _inv[None, :]
_COS = jnp.asarray(np.repeat(np.cos(_pos), 2, axis=-1), dtype=jnp.float32)
_SIN = jnp.asarray(np.repeat(np.sin(_pos), 2, axis=-1), dtype=jnp.float32)

_Rnp = np.zeros((DH, DH), dtype=np.float32)
for _k in range(DH // 2):
    _Rnp[2 * _k + 1, 2 * _k] = -1.0
    _Rnp[2 * _k, 2 * _k + 1] = 1.0
_ROT = jnp.asarray(_Rnp)


def _qkv_body(x_ref, wq_ref, wk_ref, wv_ref, cos_ref, sin_ref, rot_ref,
              q_ref, k_ref, v_ref):
    x = x_ref[...].astype(jnp.bfloat16)
    rot = rot_ref[...].astype(jnp.bfloat16)
    cos = cos_ref[...]
    sin = sin_ref[...]

    def rope(t):
        tr = jnp.dot(t.astype(jnp.bfloat16), rot,
                     preferred_element_type=jnp.float32)
        return t * cos + tr * sin

    q = jnp.dot(x, wq_ref[...].astype(jnp.bfloat16),
                preferred_element_type=jnp.float32)
    k = jnp.dot(x, wk_ref[...].astype(jnp.bfloat16),
                preferred_element_type=jnp.float32)
    q_ref[...] = rope(q)
    k_ref[...] = rope(k)
    v_ref[...] = jnp.dot(x, wv_ref[...].astype(jnp.bfloat16),
                         preferred_element_type=jnp.float32)


def _qkv(x2, Wq, Wk, Wv):
    w_spec = pl.BlockSpec((D, DH), lambda h: (0, h))
    t_spec = pl.BlockSpec((SQ, DH), lambda h: (0, 0))
    o_spec = pl.BlockSpec((SQ, DH), lambda h: (0, h))
    o_shape = jax.ShapeDtypeStruct((SQ, D), jnp.float32)
    return pl.pallas_call(
        _qkv_body,
        grid=(H_LOC,),
        in_specs=[
            pl.BlockSpec((SQ, D), lambda h: (0, 0)),
            w_spec, w_spec, w_spec,
            t_spec, t_spec,
            pl.BlockSpec((DH, DH), lambda h: (0, 0)),
        ],
        out_specs=[o_spec, o_spec, o_spec],
        out_shape=[o_shape, o_shape, o_shape],
    )(x2, Wq, Wk, Wv, _COS, _SIN, _ROT)


def _attn_body(q_ref, k_ref, v_ref, o_ref):
    s = lax.dot_general(
        q_ref[...].astype(jnp.bfloat16), k_ref[...].astype(jnp.bfloat16),
        (((1,), (1,)), ((), ())),
        preferred_element_type=jnp.float32,
    ) * SCALE
    w = jnp.exp(s)
    denom = jnp.sum(w, axis=-1, keepdims=True)
    o_ref[...] = jnp.dot(w.astype(jnp.bfloat16),
                         v_ref[...].astype(jnp.bfloat16),
                         preferred_element_type=jnp.float32) / denom


def _attention(q, k, v):
    kv_spec = pl.BlockSpec((SQ, DH), lambda h, qb: (0, h))
    return pl.pallas_call(
        _attn_body,
        grid=(H_LOC, SQ // QB),
        in_specs=[
            pl.BlockSpec((QB, DH), lambda h, qb: (qb, h)),
            kv_spec,
            kv_spec,
        ],
        out_specs=pl.BlockSpec((QB, DH), lambda h, qb: (qb, h)),
        out_shape=jax.ShapeDtypeStruct((SQ, D), jnp.float32),
    )(q, k, v)


_RS_BASE = (0, 4, 6)
_AG_BASE = {2: 7, 1: 8, 0: 10}


def _proj_ar_body(ctx_ref, wo_ref, o_ref, comm_ref, p16_ref,
                  send_sems, recv_sems):
    me = lax.axis_index("i")

    wo = wo_ref[...].astype(jnp.bfloat16)
    for c in range(N_DEV):
        p = jnp.dot(
            ctx_ref[pl.ds(c * CHUNK, CHUNK), :].astype(jnp.bfloat16), wo,
            preferred_element_type=jnp.float32,
        )
        o_ref[c] = p
        p16_ref[c] = p.astype(jnp.bfloat16)

    barrier = pltpu.get_barrier_semaphore()
    for s in range(3):
        pl.semaphore_signal(
            barrier, inc=1, device_id=(me ^ (1 << s),),
            device_id_type=pl.DeviceIdType.MESH,
        )
    pl.semaphore_wait(barrier, 3)

    for s in range(3):
        partner = me ^ (1 << s)
        low = me & ((1 << s) - 1)
        mebit = (me >> s) & 1
        pbit = 1 - mebit
        rdmas = []
        for j in range(N_DEV >> (s + 1)):
            c_send = j * (1 << (s + 1)) + pbit * (1 << s) + low
            slot = _RS_BASE[s] + j
            rdma = pltpu.make_async_remote_copy(
                src_ref=p16_ref.at[c_send],
                dst_ref=comm_ref.at[slot],
                send_sem=send_sems.at[slot],
                recv_sem=recv_sems.at[slot],
                device_id=(partner,),
                device_id_type=pl.DeviceIdType.MESH,
            )
            rdma.start()
            rdmas.append(rdma)
        for rdma in rdmas:
            rdma.wait()
        for j in range(N_DEV >> (s + 1)):
            c_keep = j * (1 << (s + 1)) + mebit * (1 << s) + low
            slot = _RS_BASE[s] + j
            acc = o_ref[c_keep] + comm_ref[slot].astype(jnp.float32)
            o_ref[c_keep] = acc
            p16_ref[c_keep] = acc.astype(jnp.bfloat16)

    for s in (2, 1, 0):
        partner = me ^ (1 << s)
        lowmask = (1 << (s + 1)) - 1
        mylow = me & lowmask
        plow = partner & lowmask
        rdmas = []
        for j in range(N_DEV >> (s + 1)):
            c_send = j * (1 << (s + 1)) + mylow
            slot = _AG_BASE[s] + j
            rdma = pltpu.make_async_remote_copy(
                src_ref=p16_ref.at[c_send],
                dst_ref=comm_ref.at[slot],
                send_sem=send_sems.at[slot],
                recv_sem=recv_sems.at[slot],
                device_id=(partner,),
                device_id_type=pl.DeviceIdType.MESH,
            )
            rdma.start()
            rdmas.append(rdma)
        for rdma in rdmas:
            rdma.wait()
        for j in range(N_DEV >> (s + 1)):
            c_recv = j * (1 << (s + 1)) + plow
            slot = _AG_BASE[s] + j
            o_ref[c_recv] = comm_ref[slot].astype(jnp.float32)
            p16_ref[c_recv] = comm_ref[slot]


def _proj_allreduce(ctx, Wo):
    n_sems = 2 * (N_DEV - 1)
    return pl.pallas_call(
        _proj_ar_body,
        in_specs=[
            pl.BlockSpec(memory_space=pltpu.VMEM),
            pl.BlockSpec(memory_space=pltpu.VMEM),
        ],
        out_specs=pl.BlockSpec(memory_space=pltpu.VMEM),
        out_shape=jax.ShapeDtypeStruct((N_DEV, CHUNK, D), jnp.float32),
        scratch_shapes=[
            pltpu.VMEM((2 * (N_DEV - 1), CHUNK, D), jnp.bfloat16),
            pltpu.VMEM((N_DEV, CHUNK, D), jnp.bfloat16),
            pltpu.SemaphoreType.DMA((n_sems,)),
            pltpu.SemaphoreType.DMA((n_sems,)),
        ],
        compiler_params=pltpu.CompilerParams(collective_id=0),
    )(ctx, Wo)


def kernel(x, Wq, Wk, Wv, Wo):
    x2 = x.reshape(SQ, D)
    q, k, v = _qkv(x2, Wq, Wk, Wv)
    ctx = _attention(q, k, v)
    out = _proj_allreduce(ctx, Wo)
    return out.reshape(1, SQ, D)


# device time: 194807 ns/iter; 2.2705x vs baseline; 1.0566x over previous
import numpy as np

import jax
import jax.numpy as jnp
from jax import lax
from jax.experimental import pallas as pl
from jax.experimental.pallas import tpu as pltpu

N_DEV = 8
SQ = 2048
D = 1024
DH = 128
H_LOC = D // DH
QB = 512
CHUNK = SQ // N_DEV
SCALE = 0.08838834764831843

_inv = 1.0 / (10000.0 ** (np.arange(0, DH, 2) / DH))
_pos = np.arange(SQ)[:, None] * _inv[None, :]
_COS = jnp.asarray(np.repeat(np.cos(_pos), 2, axis=-1), dtype=jnp.float32)
_SIN = jnp.asarray(np.repeat(np.sin(_pos), 2, axis=-1), dtype=jnp.float32)

_Rnp = np.zeros((DH, DH), dtype=np.float32)
for _k in range(DH // 2):
    _Rnp[2 * _k + 1, 2 * _k] = -1.0
    _Rnp[2 * _k, 2 * _k + 1] = 1.0
_ROT = jnp.asarray(_Rnp)


def _qkv_body(x_ref, wq_ref, wk_ref, wv_ref, cos_ref, sin_ref, rot_ref,
              q_ref, k_ref, v_ref):
    x = x_ref[...].astype(jnp.bfloat16)
    rot = rot_ref[...].astype(jnp.bfloat16)
    cos = cos_ref[...]
    sin = sin_ref[...]

    def rope(t):
        tr = jnp.dot(t.astype(jnp.bfloat16), rot,
                     preferred_element_type=jnp.float32)
        return t * cos + tr * sin

    q = jnp.dot(x, wq_ref[...].astype(jnp.bfloat16),
                preferred_element_type=jnp.float32)
    k = jnp.dot(x, wk_ref[...].astype(jnp.bfloat16),
                preferred_element_type=jnp.float32)
    q_ref[...] = rope(q)
    k_ref[...] = rope(k)
    v_ref[...] = jnp.dot(x, wv_ref[...].astype(jnp.bfloat16),
                         preferred_element_type=jnp.float32)


def _qkv(x2, Wq, Wk, Wv):
    w_spec = pl.BlockSpec((D, DH), lambda h: (0, h))
    t_spec = pl.BlockSpec((SQ, DH), lambda h: (0, 0))
    o_spec = pl.BlockSpec((SQ, DH), lambda h: (0, h))
    o_shape = jax.ShapeDtypeStruct((SQ, D), jnp.float32)
    return pl.pallas_call(
        _qkv_body,
        grid=(H_LOC,),
        in_specs=[
            pl.BlockSpec((SQ, D), lambda h: (0, 0)),
            w_spec, w_spec, w_spec,
            t_spec, t_spec,
            pl.BlockSpec((DH, DH), lambda h: (0, 0)),
        ],
        out_specs=[o_spec, o_spec, o_spec],
        out_shape=[o_shape, o_shape, o_shape],
    )(x2, Wq, Wk, Wv, _COS, _SIN, _ROT)


def _attn_body(q_ref, k_ref, v_ref, o_ref):
    s = lax.dot_general(
        q_ref[...].astype(jnp.bfloat16), k_ref[...].astype(jnp.bfloat16),
        (((1,), (1,)), ((), ())),
        preferred_element_type=jnp.float32,
    ) * SCALE
    w = jnp.exp(s)
    denom = jnp.sum(w, axis=-1, keepdims=True)
    o_ref[...] = jnp.dot(w.astype(jnp.bfloat16),
                         v_ref[...].astype(jnp.bfloat16),
                         preferred_element_type=jnp.float32) / denom


def _attention(q, k, v):
    kv_spec = pl.BlockSpec((SQ, DH), lambda h, qb: (0, h))
    return pl.pallas_call(
        _attn_body,
        grid=(H_LOC, SQ // QB),
        in_specs=[
            pl.BlockSpec((QB, DH), lambda h, qb: (qb, h)),
            kv_spec,
            kv_spec,
        ],
        out_specs=pl.BlockSpec((QB, DH), lambda h, qb: (qb, h)),
        out_shape=jax.ShapeDtypeStruct((SQ, D), jnp.float32),
    )(q, k, v)


_RS_BASE = (0, 4, 6)
_AG_BASE = {2: 7, 1: 8, 0: 10}
_HALF = D // 2
_BIT_ORDER = ((0, 1, 2), (2, 1, 0))


def _proj_ar_body(ctx_ref, wo_ref, o_ref, comm_ref, p16_ref,
                  send_sems, recv_sems):
    me = lax.axis_index("i")

    wo = wo_ref[...].astype(jnp.bfloat16)
    for c in range(N_DEV):
        p = jnp.dot(
            ctx_ref[pl.ds(c * CHUNK, CHUNK), :].astype(jnp.bfloat16), wo,
            preferred_element_type=jnp.float32,
        )
        o_ref[c] = p
        for b in range(2):
            p16_ref[b, c] = p[:, b * _HALF:(b + 1) * _HALF].astype(jnp.bfloat16)

    barrier = pltpu.get_barrier_semaphore()
    for s in range(3):
        pl.semaphore_signal(
            barrier, inc=1, device_id=(me ^ (1 << s),),
            device_id_type=pl.DeviceIdType.MESH,
        )
    pl.semaphore_wait(barrier, 3)

    for stage_i in range(3):
        rdmas = []
        for b in range(2):
            s = _BIT_ORDER[b][stage_i]
            partner = me ^ (1 << s)
            low = me & ((1 << s) - 1)
            pbit = 1 - ((me >> s) & 1)
            for j in range(N_DEV >> (s + 1)):
                c_send = j * (1 << (s + 1)) + pbit * (1 << s) + low
                slot = _RS_BASE[s] + j
                rdma = pltpu.make_async_remote_copy(
                    src_ref=p16_ref.at[b, c_send],
                    dst_ref=comm_ref.at[b, slot],
                    send_sem=send_sems.at[b, slot],
                    recv_sem=recv_sems.at[b, slot],
                    device_id=(partner,),
                    device_id_type=pl.DeviceIdType.MESH,
                )
                rdma.start()
                rdmas.append(rdma)
        for rdma in rdmas:
            rdma.wait()
        for b in range(2):
            s = _BIT_ORDER[b][stage_i]
            low = me & ((1 << s) - 1)
            mebit = (me >> s) & 1
            for j in range(N_DEV >> (s + 1)):
                c_keep = j * (1 << (s + 1)) + mebit * (1 << s) + low
                slot = _RS_BASE[s] + j
                acc = (o_ref[c_keep, :, pl.ds(b * _HALF, _HALF)]
                       + comm_ref[b, slot].astype(jnp.float32))
                o_ref[c_keep, :, pl.ds(b * _HALF, _HALF)] = acc
                p16_ref[b, c_keep] = acc.astype(jnp.bfloat16)

    for stage_i in range(3):
        rdmas = []
        for b in range(2):
            s = _BIT_ORDER[b][2 - stage_i]
            mylow = me & ((1 << (s + 1)) - 1)
            for j in range(N_DEV >> (s + 1)):
                c_send = j * (1 << (s + 1)) + mylow
                slot = _AG_BASE[s] + j
                rdma = pltpu.make_async_remote_copy(
                    src_ref=p16_ref.at[b, c_send],
                    dst_ref=comm_ref.at[b, slot],
                    send_sem=send_sems.at[b, slot],
                    recv_sem=recv_sems.at[b, slot],
                    device_id=(me ^ (1 << s),),
                    device_id_type=pl.DeviceIdType.MESH,
                )
                rdma.start()
                rdmas.append(rdma)
        for rdma in rdmas:
            rdma.wait()
        for b in range(2):
            s = _BIT_ORDER[b][2 - stage_i]
            plow = (me ^ (1 << s)) & ((1 << (s + 1)) - 1)
            for j in range(N_DEV >> (s + 1)):
                c_recv = j * (1 << (s + 1)) + plow
                slot = _AG_BASE[s] + j
                o_ref[c_recv, :, pl.ds(b * _HALF, _HALF)] = (
                    comm_ref[b, slot].astype(jnp.float32))
                p16_ref[b, c_recv] = comm_ref[b, slot]


def _proj_allreduce(ctx, Wo):
    n_sems = 2 * (N_DEV - 1)
    return pl.pallas_call(
        _proj_ar_body,
        in_specs=[
            pl.BlockSpec(memory_space=pltpu.VMEM),
            pl.BlockSpec(memory_space=pltpu.VMEM),
        ],
        out_specs=pl.BlockSpec(memory_space=pltpu.VMEM),
        out_shape=jax.ShapeDtypeStruct((N_DEV, CHUNK, D), jnp.float32),
        scratch_shapes=[
            pltpu.VMEM((2, n_sems, CHUNK, _HALF), jnp.bfloat16),
            pltpu.VMEM((2, N_DEV, CHUNK, _HALF), jnp.bfloat16),
            pltpu.SemaphoreType.DMA((2, n_sems)),
            pltpu.SemaphoreType.DMA((2, n_sems)),
        ],
        compiler_params=pltpu.CompilerParams(collective_id=0),
    )(ctx, Wo)


def kernel(x, Wq, Wk, Wv, Wo):
    x2 = x.reshape(SQ, D)
    q, k, v = _qkv(x2, Wq, Wk, Wv)
    ctx = _attention(q, k, v)
    out = _proj_allreduce(ctx, Wo)
    return out.reshape(1, SQ, D)


# device time: 178918 ns/iter; 2.4722x vs baseline; 1.0888x over previous
import numpy as np

import jax
import jax.numpy as jnp
from jax import lax
from jax.experimental import pallas as pl
from jax.experimental.pallas import tpu as pltpu

N_DEV = 8
SQ = 2048
D = 1024
DH = 128
H_LOC = D // DH
QB = 512
CHUNK = SQ // N_DEV
SCALE = 0.08838834764831843

_inv = 1.0 / (10000.0 ** (np.arange(0, DH, 2) / DH))
_pos = np.arange(SQ)[:, None] * _inv[None, :]
_COS = jnp.asarray(np.repeat(np.cos(_pos), 2, axis=-1), dtype=jnp.float32)
_SIN = jnp.asarray(np.repeat(np.sin(_pos), 2, axis=-1), dtype=jnp.float32)

_Rnp = np.zeros((DH, DH), dtype=np.float32)
for _k in range(DH // 2):
    _Rnp[2 * _k + 1, 2 * _k] = -1.0
    _Rnp[2 * _k, 2 * _k + 1] = 1.0
_ROT = jnp.asarray(_Rnp)


def _qkv_body(x_ref, wq_ref, wk_ref, wv_ref, cos_ref, sin_ref, rot_ref,
              q_ref, k_ref, v_ref):
    x = x_ref[...].astype(jnp.bfloat16)
    rot = rot_ref[...].astype(jnp.bfloat16)
    cos = cos_ref[...]
    sin = sin_ref[...]

    def rope(t):
        tr = jnp.dot(t.astype(jnp.bfloat16), rot,
                     preferred_element_type=jnp.float32)
        return t * cos + tr * sin

    q = jnp.dot(x, wq_ref[...].astype(jnp.bfloat16),
                preferred_element_type=jnp.float32)
    k = jnp.dot(x, wk_ref[...].astype(jnp.bfloat16),
                preferred_element_type=jnp.float32)
    q_ref[...] = rope(q)
    k_ref[...] = rope(k)
    v_ref[...] = jnp.dot(x, wv_ref[...].astype(jnp.bfloat16),
                         preferred_element_type=jnp.float32)


def _qkv(x2, Wq, Wk, Wv):
    w_spec = pl.BlockSpec((D, DH), lambda h: (0, h))
    t_spec = pl.BlockSpec((SQ, DH), lambda h: (0, 0))
    o_spec = pl.BlockSpec((SQ, DH), lambda h: (0, h))
    o_shape = jax.ShapeDtypeStruct((SQ, D), jnp.float32)
    return pl.pallas_call(
        _qkv_body,
        grid=(H_LOC,),
        in_specs=[
            pl.BlockSpec((SQ, D), lambda h: (0, 0)),
            w_spec, w_spec, w_spec,
            t_spec, t_spec,
            pl.BlockSpec((DH, DH), lambda h: (0, 0)),
        ],
        out_specs=[o_spec, o_spec, o_spec],
        out_shape=[o_shape, o_shape, o_shape],
    )(x2, Wq, Wk, Wv, _COS, _SIN, _ROT)


def _attn_body(q_ref, k_ref, v_ref, o_ref):
    s = lax.dot_general(
        q_ref[...].astype(jnp.bfloat16), k_ref[...].astype(jnp.bfloat16),
        (((1,), (1,)), ((), ())),
        preferred_element_type=jnp.float32,
    ) * SCALE
    w = jnp.exp(s)
    denom = jnp.sum(w, axis=-1, keepdims=True)
    o_ref[...] = jnp.dot(w.astype(jnp.bfloat16),
                         v_ref[...].astype(jnp.bfloat16),
                         preferred_element_type=jnp.float32) / denom


def _attention(q, k, v):
    kv_spec = pl.BlockSpec((SQ, DH), lambda h, qb: (0, h))
    return pl.pallas_call(
        _attn_body,
        grid=(H_LOC, SQ // QB),
        in_specs=[
            pl.BlockSpec((QB, DH), lambda h, qb: (qb, h)),
            kv_spec,
            kv_spec,
        ],
        out_specs=pl.BlockSpec((QB, DH), lambda h, qb: (qb, h)),
        out_shape=jax.ShapeDtypeStruct((SQ, D), jnp.float32),
    )(q, k, v)


_RS_BASE = (0, 4, 6)
_AG_BASE = (7, 8, 10)
_HALF = D // 2
_BIT_ORDER = ((0, 1, 2), (2, 1, 0))


def _rs_chunks(b, stage_i, me):
    s = _BIT_ORDER[b][stage_i]
    mebit = (me >> s) & 1
    pbit = 1 - mebit
    out = []
    for j in range(4 >> stage_i):
        if b == 0:
            base = me & ((1 << s) - 1)
            c_send = (j << (s + 1)) | (pbit << s) | base
            c_keep = (j << (s + 1)) | (mebit << s) | base
        else:
            base = me & (7 ^ ((1 << (s + 1)) - 1))
            c_send = base | (pbit << s) | j
            c_keep = base | (mebit << s) | j
        out.append((c_send, c_keep, _RS_BASE[stage_i] + j))
    return out


def _ag_chunks(b, k, me):
    s = _BIT_ORDER[b][2 - k]
    partner_bit = 1 << s
    out = []
    for j in range(1 << k):
        if b == 0:
            c_send = (me & ((1 << (s + 1)) - 1)) | (j << (s + 1))
            c_recv = ((me ^ partner_bit) & ((1 << (s + 1)) - 1)) | (j << (s + 1))
        else:
            mask = 7 ^ ((1 << k) - 1)
            c_send = (me & mask) | j
            c_recv = ((me ^ partner_bit) & mask) | j
        out.append((c_send, c_recv, _AG_BASE[k] + j, s))
    return out


def _proj_ar_body(ctx_ref, wo_ref, o_ref, comm_ref, p16_ref,
                  send_sems, recv_sems):
    me = lax.axis_index("i")

    wo = wo_ref[...].astype(jnp.bfloat16)
    for c in range(N_DEV):
        p = jnp.dot(
            ctx_ref[pl.ds(c * CHUNK, CHUNK), :].astype(jnp.bfloat16), wo,
            preferred_element_type=jnp.float32,
        )
        o_ref[c] = p
        for b in range(2):
            p16_ref[b, c] = p[:, b * _HALF:(b + 1) * _HALF].astype(jnp.bfloat16)

    barrier = pltpu.get_barrier_semaphore()
    for s in range(3):
        pl.semaphore_signal(
            barrier, inc=1, device_id=(me ^ (1 << s),),
            device_id_type=pl.DeviceIdType.MESH,
        )
    pl.semaphore_wait(barrier, 3)

    for stage_i in range(3):
        rdmas = []
        for b in range(2):
            s = _BIT_ORDER[b][stage_i]
            partner = me ^ (1 << s)
            for c_send, _, slot in _rs_chunks(b, stage_i, me):
                rdma = pltpu.make_async_remote_copy(
                    src_ref=p16_ref.at[b, c_send],
                    dst_ref=comm_ref.at[b, slot],
                    send_sem=send_sems.at[b, slot],
                    recv_sem=recv_sems.at[b, slot],
                    device_id=(partner,),
                    device_id_type=pl.DeviceIdType.MESH,
                )
                rdma.start()
                rdmas.append(rdma)
        for rdma in rdmas:
            rdma.wait()
        for b in range(2):
            for _, c_keep, slot in _rs_chunks(b, stage_i, me):
                acc = (o_ref[c_keep, :, pl.ds(b * _HALF, _HALF)]
                       + comm_ref[b, slot].astype(jnp.float32))
                o_ref[c_keep, :, pl.ds(b * _HALF, _HALF)] = acc
                p16_ref[b, c_keep] = acc.astype(jnp.bfloat16)

    for k in range(3):
        rdmas = []
        for b in range(2):
            for c_send, _, slot, s in _ag_chunks(b, k, me):
                rdma = pltpu.make_async_remote_copy(
                    src_ref=p16_ref.at[b, c_send],
                    dst_ref=comm_ref.at[b, slot],
                    send_sem=send_sems.at[b, slot],
                    recv_sem=recv_sems.at[b, slot],
                    device_id=(me ^ (1 << s),),
                    device_id_type=pl.DeviceIdType.MESH,
                )
                rdma.start()
                rdmas.append(rdma)
        for rdma in rdmas:
            rdma.wait()
        for b in range(2):
            for _, c_recv, slot, _s in _ag_chunks(b, k, me):
                o_ref[c_recv, :, pl.ds(b * _HALF, _HALF)] = (
                    comm_ref[b, slot].astype(jnp.float32))
                p16_ref[b, c_recv] = comm_ref[b, slot]


def _proj_allreduce(ctx, Wo):
    n_sems = 2 * (N_DEV - 1)
    return pl.pallas_call(
        _proj_ar_body,
        in_specs=[
            pl.BlockSpec(memory_space=pltpu.VMEM),
            pl.BlockSpec(memory_space=pltpu.VMEM),
        ],
        out_specs=pl.BlockSpec(memory_space=pltpu.VMEM),
        out_shape=jax.ShapeDtypeStruct((N_DEV, CHUNK, D), jnp.float32),
        scratch_shapes=[
            pltpu.VMEM((2, n_sems, CHUNK, _HALF), jnp.bfloat16),
            pltpu.VMEM((2, N_DEV, CHUNK, _HALF), jnp.bfloat16),
            pltpu.SemaphoreType.DMA((2, n_sems)),
            pltpu.SemaphoreType.DMA((2, n_sems)),
        ],
        compiler_params=pltpu.CompilerParams(collective_id=0),
    )(ctx, Wo)


def kernel(x, Wq, Wk, Wv, Wo):
    x2 = x.reshape(SQ, D)
    q, k, v = _qkv(x2, Wq, Wk, Wv)
    ctx = _attention(q, k, v)
    out = _proj_allreduce(ctx, Wo)
    return out.reshape(1, SQ, D)


# device time: 173952 ns/iter; 2.5427x vs baseline; 1.0285x over previous
import numpy as np

import jax
import jax.numpy as jnp
from jax import lax
from jax.experimental import pallas as pl
from jax.experimental.pallas import tpu as pltpu

N_DEV = 8
SQ = 2048
D = 1024
DH = 128
H_LOC = D // DH
QB = 512
CHUNK = SQ // N_DEV
SCALE = 0.08838834764831843

_inv = 1.0 / (10000.0 ** (np.arange(0, DH, 2) / DH))
_pos = np.arange(SQ)[:, None] * _inv[None, :]
_COS = jnp.asarray(np.repeat(np.cos(_pos), 2, axis=-1), dtype=jnp.float32)
_SIN = jnp.asarray(np.repeat(np.sin(_pos), 2, axis=-1), dtype=jnp.float32)

_Rnp = np.zeros((DH, DH), dtype=np.float32)
for _k in range(DH // 2):
    _Rnp[2 * _k + 1, 2 * _k] = -1.0
    _Rnp[2 * _k, 2 * _k + 1] = 1.0
_ROT = jnp.asarray(_Rnp)


def _qkv_body(x_ref, wq_ref, wk_ref, wv_ref, cos_ref, sin_ref, rot_ref,
              q_ref, k_ref, v_ref):
    x = x_ref[...].astype(jnp.bfloat16)
    rot = rot_ref[...].astype(jnp.bfloat16)
    cos = cos_ref[...]
    sin = sin_ref[...]

    def rope(t):
        tr = jnp.dot(t.astype(jnp.bfloat16), rot,
                     preferred_element_type=jnp.float32)
        return t * cos + tr * sin

    q = jnp.dot(x, wq_ref[...].astype(jnp.bfloat16),
                preferred_element_type=jnp.float32)
    k = jnp.dot(x, wk_ref[...].astype(jnp.bfloat16),
                preferred_element_type=jnp.float32)
    q_ref[...] = rope(q)
    k_ref[...] = rope(k)
    v_ref[...] = jnp.dot(x, wv_ref[...].astype(jnp.bfloat16),
                         preferred_element_type=jnp.float32)


def _qkv(x2, Wq, Wk, Wv):
    w_spec = pl.BlockSpec((D, DH), lambda h: (0, h))
    t_spec = pl.BlockSpec((SQ, DH), lambda h: (0, 0))
    o_spec = pl.BlockSpec((SQ, DH), lambda h: (0, h))
    o_shape = jax.ShapeDtypeStruct((SQ, D), jnp.float32)
    return pl.pallas_call(
        _qkv_body,
        grid=(H_LOC,),
        in_specs=[
            pl.BlockSpec((SQ, D), lambda h: (0, 0)),
            w_spec, w_spec, w_spec,
            t_spec, t_spec,
            pl.BlockSpec((DH, DH), lambda h: (0, 0)),
        ],
        out_specs=[o_spec, o_spec, o_spec],
        out_shape=[o_shape, o_shape, o_shape],
    )(x2, Wq, Wk, Wv, _COS, _SIN, _ROT)


def _attn_body(q_ref, k_ref, v_ref, o_ref):
    s = lax.dot_general(
        q_ref[...].astype(jnp.bfloat16), k_ref[...].astype(jnp.bfloat16),
        (((1,), (1,)), ((), ())),
        preferred_element_type=jnp.float32,
    ) * SCALE
    w = jnp.exp(s)
    denom = jnp.sum(w, axis=-1, keepdims=True)
    o_ref[...] = jnp.dot(w.astype(jnp.bfloat16),
                         v_ref[...].astype(jnp.bfloat16),
                         preferred_element_type=jnp.float32) / denom


def _attention(q, k, v):
    kv_spec = pl.BlockSpec((SQ, DH), lambda h, qb: (0, h))
    return pl.pallas_call(
        _attn_body,
        grid=(H_LOC, SQ // QB),
        in_specs=[
            pl.BlockSpec((QB, DH), lambda h, qb: (qb, h)),
            kv_spec,
            kv_spec,
        ],
        out_specs=pl.BlockSpec((QB, DH), lambda h, qb: (qb, h)),
        out_shape=jax.ShapeDtypeStruct((SQ, D), jnp.float32),
    )(q, k, v)


_RS_BASE = (0, 4, 6)
_AG_BASE = (7, 8, 10)
_HALF = D // 2
_BIT_ORDER = ((0, 1, 2), (2, 1, 0))


def _rs_chunks(b, stage_i, me):
    s = _BIT_ORDER[b][stage_i]
    mebit = (me >> s) & 1
    pbit = 1 - mebit
    out = []
    for j in range(4 >> stage_i):
        if b == 0:
            base = me & ((1 << s) - 1)
            c_send = (j << (s + 1)) | (pbit << s) | base
            c_keep = (j << (s + 1)) | (mebit << s) | base
        else:
            base = me & (7 ^ ((1 << (s + 1)) - 1))
            c_send = base | (pbit << s) | j
            c_keep = base | (mebit << s) | j
        out.append((c_send, c_keep, _RS_BASE[stage_i] + j))
    return out


def _ag_chunks(b, k, me):
    s = _BIT_ORDER[b][2 - k]
    partner_bit = 1 << s
    out = []
    for j in range(1 << k):
        if b == 0:
            c_send = (me & ((1 << (s + 1)) - 1)) | (j << (s + 1))
            c_recv = ((me ^ partner_bit) & ((1 << (s + 1)) - 1)) | (j << (s + 1))
        else:
            mask = 7 ^ ((1 << k) - 1)
            c_send = (me & mask) | j
            c_recv = ((me ^ partner_bit) & mask) | j
        out.append((c_send, c_recv, _AG_BASE[k] + j, s))
    return out


def _proj_ar_body(ctx_ref, wo_ref, o_ref, comm_ref, p16_ref,
                  send_sems, recv_sems):
    me = lax.axis_index("i")

    barrier = pltpu.get_barrier_semaphore()
    for s in range(3):
        pl.semaphore_signal(
            barrier, inc=1, device_id=(me ^ (1 << s),),
            device_id_type=pl.DeviceIdType.MESH,
        )

    wo = wo_ref[...].astype(jnp.bfloat16)
    for c in range(N_DEV):
        p = jnp.dot(
            ctx_ref[pl.ds(c * CHUNK, CHUNK), :].astype(jnp.bfloat16), wo,
            preferred_element_type=jnp.float32,
        )
        o_ref[c] = p
        for b in range(2):
            p16_ref[b, c] = p[:, b * _HALF:(b + 1) * _HALF].astype(jnp.bfloat16)

    pl.semaphore_wait(barrier, 3)

    def rs_start(b, stage_i):
        s = _BIT_ORDER[b][stage_i]
        partner = me ^ (1 << s)
        rdmas = []
        for c_send, _, slot in _rs_chunks(b, stage_i, me):
            rdma = pltpu.make_async_remote_copy(
                src_ref=p16_ref.at[b, c_send],
                dst_ref=comm_ref.at[b, slot],
                send_sem=send_sems.at[b, slot],
                recv_sem=recv_sems.at[b, slot],
                device_id=(partner,),
                device_id_type=pl.DeviceIdType.MESH,
            )
            rdma.start()
            rdmas.append(rdma)
        return rdmas

    def rs_fin(b, stage_i, rdmas):
        for rdma in rdmas:
            rdma.wait()
        for _, c_keep, slot in _rs_chunks(b, stage_i, me):
            acc = (o_ref[c_keep, :, pl.ds(b * _HALF, _HALF)]
                   + comm_ref[b, slot].astype(jnp.float32))
            o_ref[c_keep, :, pl.ds(b * _HALF, _HALF)] = acc
            p16_ref[b, c_keep] = acc.astype(jnp.bfloat16)

    def ag_start(b, k):
        rdmas = []
        for c_send, _, slot, s in _ag_chunks(b, k, me):
            rdma = pltpu.make_async_remote_copy(
                src_ref=p16_ref.at[b, c_send],
                dst_ref=comm_ref.at[b, slot],
                send_sem=send_sems.at[b, slot],
                recv_sem=recv_sems.at[b, slot],
                device_id=(me ^ (1 << s),),
                device_id_type=pl.DeviceIdType.MESH,
            )
            rdma.start()
            rdmas.append(rdma)
        return rdmas

    def ag_fin(b, k, rdmas):
        for rdma in rdmas:
            rdma.wait()
        for _, c_recv, slot, _s in _ag_chunks(b, k, me):
            o_ref[c_recv, :, pl.ds(b * _HALF, _HALF)] = (
                comm_ref[b, slot].astype(jnp.float32))
            if k < 2:
                p16_ref[b, c_recv] = comm_ref[b, slot]

    a = rs_start(0, 0)
    bb = rs_start(1, 0)
    rs_fin(0, 0, a);  a = rs_start(0, 1)
    rs_fin(1, 0, bb); bb = rs_start(1, 1)
    rs_fin(0, 1, a);  a = rs_start(0, 2)
    rs_fin(1, 1, bb); bb = rs_start(1, 2)
    rs_fin(0, 2, a);  a = ag_start(0, 0)
    rs_fin(1, 2, bb); bb = ag_start(1, 0)
    ag_fin(0, 0, a);  a = ag_start(0, 1)
    ag_fin(1, 0, bb); bb = ag_start(1, 1)
    ag_fin(0, 1, a);  a = ag_start(0, 2)
    ag_fin(1, 1, bb); bb = ag_start(1, 2)
    ag_fin(0, 2, a)
    ag_fin(1, 2, bb)


def _proj_allreduce(ctx, Wo):
    n_sems = 2 * (N_DEV - 1)
    return pl.pallas_call(
        _proj_ar_body,
        in_specs=[
            pl.BlockSpec(memory_space=pltpu.VMEM),
            pl.BlockSpec(memory_space=pltpu.VMEM),
        ],
        out_specs=pl.BlockSpec(memory_space=pltpu.VMEM),
        out_shape=jax.ShapeDtypeStruct((N_DEV, CHUNK, D), jnp.float32),
        scratch_shapes=[
            pltpu.VMEM((2, n_sems, CHUNK, _HALF), jnp.bfloat16),
            pltpu.VMEM((2, N_DEV, CHUNK, _HALF), jnp.bfloat16),
            pltpu.SemaphoreType.DMA((2, n_sems)),
            pltpu.SemaphoreType.DMA((2, n_sems)),
        ],
        compiler_params=pltpu.CompilerParams(collective_id=0),
    )(ctx, Wo)


def kernel(x, Wq, Wk, Wv, Wo):
    x2 = x.reshape(SQ, D)
    q, k, v = _qkv(x2, Wq, Wk, Wv)
    ctx = _attention(q, k, v)
    out = _proj_allreduce(ctx, Wo)
    return out.reshape(1, SQ, D)


# device time: 159207 ns/iter; 2.7782x vs baseline; 1.0926x over previous
import numpy as np

import jax
import jax.numpy as jnp
from jax import lax
from jax.experimental import pallas as pl
from jax.experimental.pallas import tpu as pltpu

N_DEV = 8
SQ = 2048
D = 1024
DH = 128
H_LOC = D // DH
QB = 512
CHUNK = SQ // N_DEV
SCALE = 0.08838834764831843

_inv = 1.0 / (10000.0 ** (np.arange(0, DH, 2) / DH))
_pos = np.arange(SQ)[:, None] * _inv[None, :]
_COS = jnp.asarray(np.repeat(np.cos(_pos), 2, axis=-1), dtype=jnp.float32)
_SIN = jnp.asarray(np.repeat(np.sin(_pos), 2, axis=-1), dtype=jnp.float32)

_Rnp = np.zeros((DH, DH), dtype=np.float32)
for _k in range(DH // 2):
    _Rnp[2 * _k + 1, 2 * _k] = -1.0
    _Rnp[2 * _k, 2 * _k + 1] = 1.0
_ROT = jnp.asarray(_Rnp)


def _qkv_attn_body(x_ref, wq_ref, wk_ref, wv_ref, cos_ref, sin_ref, rot_ref,
                   o_ref):
    x = x_ref[...]
    rot = rot_ref[...].astype(jnp.bfloat16)
    cos = cos_ref[...]
    sin = sin_ref[...]

    def rope(t):
        tr = jnp.dot(t.astype(jnp.bfloat16), rot,
                     preferred_element_type=jnp.float32)
        return t * cos + tr * sin

    q = jnp.dot(x, wq_ref[...].astype(jnp.bfloat16),
                preferred_element_type=jnp.float32)
    k = jnp.dot(x, wk_ref[...].astype(jnp.bfloat16),
                preferred_element_type=jnp.float32)
    v = jnp.dot(x, wv_ref[...].astype(jnp.bfloat16),
                preferred_element_type=jnp.float32)
    q16 = rope(q).astype(jnp.bfloat16)
    k16 = rope(k).astype(jnp.bfloat16)
    v16 = v.astype(jnp.bfloat16)

    for i in range(SQ // QB):
        s = lax.dot_general(
            q16[i * QB:(i + 1) * QB], k16, (((1,), (1,)), ((), ())),
            preferred_element_type=jnp.float32,
        ) * SCALE
        w = jnp.exp(s)
        denom = jnp.sum(w, axis=-1, keepdims=True)
        o_ref[pl.ds(i * QB, QB), :] = jnp.dot(
            w.astype(jnp.bfloat16), v16,
            preferred_element_type=jnp.float32) / denom


def _qkv_attention(x16, Wq, Wk, Wv):
    w_spec = pl.BlockSpec((D, DH), lambda h: (0, h))
    t_spec = pl.BlockSpec((SQ, DH), lambda h: (0, 0))
    return pl.pallas_call(
        _qkv_attn_body,
        grid=(H_LOC,),
        in_specs=[
            pl.BlockSpec((SQ, D), lambda h: (0, 0)),
            w_spec, w_spec, w_spec,
            t_spec, t_spec,
            pl.BlockSpec((DH, DH), lambda h: (0, 0)),
        ],
        out_specs=pl.BlockSpec((SQ, DH), lambda h: (0, h)),
        out_shape=jax.ShapeDtypeStruct((SQ, D), jnp.float32),
    )(x16, Wq, Wk, Wv, _COS, _SIN, _ROT)


_RS_BASE = (0, 4, 6)
_AG_BASE = (7, 8, 10)
_HALF = D // 2
_BIT_ORDER = ((0, 1, 2), (2, 1, 0))


def _rs_chunks(b, stage_i, me):
    s = _BIT_ORDER[b][stage_i]
    mebit = (me >> s) & 1
    pbit = 1 - mebit
    out = []
    for j in range(4 >> stage_i):
        if b == 0:
            base = me & ((1 << s) - 1)
            c_send = (j << (s + 1)) | (pbit << s) | base
            c_keep = (j << (s + 1)) | (mebit << s) | base
        else:
            base = me & (7 ^ ((1 << (s + 1)) - 1))
            c_send = base | (pbit << s) | j
            c_keep = base | (mebit << s) | j
        out.append((c_send, c_keep, _RS_BASE[stage_i] + j))
    return out


def _ag_chunks(b, k, me):
    s = _BIT_ORDER[b][2 - k]
    partner_bit = 1 << s
    out = []
    for j in range(1 << k):
        if b == 0:
            c_send = (me & ((1 << (s + 1)) - 1)) | (j << (s + 1))
            c_recv = ((me ^ partner_bit) & ((1 << (s + 1)) - 1)) | (j << (s + 1))
        else:
            mask = 7 ^ ((1 << k) - 1)
            c_send = (me & mask) | j
            c_recv = ((me ^ partner_bit) & mask) | j
        out.append((c_send, c_recv, _AG_BASE[k] + j, s))
    return out


def _proj_ar_body(ctx_ref, wo_ref, o_ref, comm_ref, p16_ref,
                  send_sems, recv_sems):
    me = lax.axis_index("i")

    barrier = pltpu.get_barrier_semaphore()
    for s in range(3):
        pl.semaphore_signal(
            barrier, inc=1, device_id=(me ^ (1 << s),),
            device_id_type=pl.DeviceIdType.MESH,
        )

    wo = wo_ref[...].astype(jnp.bfloat16)
    for c in range(N_DEV):
        p = jnp.dot(
            ctx_ref[pl.ds(c * CHUNK, CHUNK), :].astype(jnp.bfloat16), wo,
            preferred_element_type=jnp.float32,
        )
        o_ref[c] = p
        for b in range(2):
            p16_ref[b, c] = p[:, b * _HALF:(b + 1) * _HALF].astype(jnp.bfloat16)

    pl.semaphore_wait(barrier, 3)

    def rs_start(b, stage_i):
        s = _BIT_ORDER[b][stage_i]
        partner = me ^ (1 << s)
        rdmas = []
        for c_send, _, slot in _rs_chunks(b, stage_i, me):
            rdma = pltpu.make_async_remote_copy(
                src_ref=p16_ref.at[b, c_send],
                dst_ref=comm_ref.at[b, slot],
                send_sem=send_sems.at[b, slot],
                recv_sem=recv_sems.at[b, slot],
                device_id=(partner,),
                device_id_type=pl.DeviceIdType.MESH,
            )
            rdma.start()
            rdmas.append(rdma)
        return rdmas

    def rs_fin(b, stage_i, rdmas):
        for rdma in rdmas:
            rdma.wait()
        for _, c_keep, slot in _rs_chunks(b, stage_i, me):
            acc = (o_ref[c_keep, :, pl.ds(b * _HALF, _HALF)]
                   + comm_ref[b, slot].astype(jnp.float32))
            o_ref[c_keep, :, pl.ds(b * _HALF, _HALF)] = acc
            p16_ref[b, c_keep] = acc.astype(jnp.bfloat16)

    def ag_start(b, k):
        rdmas = []
        for c_send, _, slot, s in _ag_chunks(b, k, me):
            rdma = pltpu.make_async_remote_copy(
                src_ref=p16_ref.at[b, c_send],
                dst_ref=comm_ref.at[b, slot],
                send_sem=send_sems.at[b, slot],
                recv_sem=recv_sems.at[b, slot],
                device_id=(me ^ (1 << s),),
                device_id_type=pl.DeviceIdType.MESH,
            )
            rdma.start()
            rdmas.append(rdma)
        return rdmas

    def ag_fin(b, k, rdmas):
        for rdma in rdmas:
            rdma.wait()
        for _, c_recv, slot, _s in _ag_chunks(b, k, me):
            o_ref[c_recv, :, pl.ds(b * _HALF, _HALF)] = (
                comm_ref[b, slot].astype(jnp.float32))
            if k < 2:
                p16_ref[b, c_recv] = comm_ref[b, slot]

    a = rs_start(0, 0)
    bb = rs_start(1, 0)
    rs_fin(0, 0, a);  a = rs_start(0, 1)
    rs_fin(1, 0, bb); bb = rs_start(1, 1)
    rs_fin(0, 1, a);  a = rs_start(0, 2)
    rs_fin(1, 1, bb); bb = rs_start(1, 2)
    rs_fin(0, 2, a);  a = ag_start(0, 0)
    rs_fin(1, 2, bb); bb = ag_start(1, 0)
    ag_fin(0, 0, a);  a = ag_start(0, 1)
    ag_fin(1, 0, bb); bb = ag_start(1, 1)
    ag_fin(0, 1, a);  a = ag_start(0, 2)
    ag_fin(1, 1, bb); bb = ag_start(1, 2)
    ag_fin(0, 2, a)
    ag_fin(1, 2, bb)


def _proj_allreduce(ctx, Wo):
    n_sems = 2 * (N_DEV - 1)
    return pl.pallas_call(
        _proj_ar_body,
        in_specs=[
            pl.BlockSpec(memory_space=pltpu.VMEM),
            pl.BlockSpec(memory_space=pltpu.VMEM),
        ],
        out_specs=pl.BlockSpec(memory_space=pltpu.VMEM),
        out_shape=jax.ShapeDtypeStruct((N_DEV, CHUNK, D), jnp.float32),
        scratch_shapes=[
            pltpu.VMEM((2, n_sems, CHUNK, _HALF), jnp.bfloat16),
            pltpu.VMEM((2, N_DEV, CHUNK, _HALF), jnp.bfloat16),
            pltpu.SemaphoreType.DMA((2, n_sems)),
            pltpu.SemaphoreType.DMA((2, n_sems)),
        ],
        compiler_params=pltpu.CompilerParams(collective_id=0),
    )(ctx, Wo)


def kernel(x, Wq, Wk, Wv, Wo):
    x16 = x.reshape(SQ, D).astype(jnp.bfloat16)
    ctx = _qkv_attention(x16, Wq, Wk, Wv)
    out = _proj_allreduce(ctx, Wo)
    return out.reshape(1, SQ, D)


# device time: 157655 ns/iter; 2.8056x vs baseline; 1.0098x over previous
import numpy as np

import jax
import jax.numpy as jnp
from jax import lax
from jax.experimental import pallas as pl
from jax.experimental.pallas import tpu as pltpu

N_DEV = 8
SQ = 2048
D = 1024
DH = 128
H_LOC = D // DH
QB = 512
CHUNK = SQ // N_DEV
SCALE = 0.08838834764831843

_inv = 1.0 / (10000.0 ** (np.arange(0, DH, 2) / DH))
_pos = np.arange(SQ)[:, None] * _inv[None, :]
_COS = jnp.asarray(np.repeat(np.cos(_pos), 2, axis=-1), dtype=jnp.float32)
_SIN = jnp.asarray(np.repeat(np.sin(_pos), 2, axis=-1), dtype=jnp.float32)

_Rnp = np.zeros((DH, DH), dtype=np.float32)
for _k in range(DH // 2):
    _Rnp[2 * _k + 1, 2 * _k] = -1.0
    _Rnp[2 * _k, 2 * _k + 1] = 1.0
_ROT = jnp.asarray(_Rnp)


def _qkv_attn_body(x_ref, wq_ref, wk_ref, wv_ref, cos_ref, sin_ref, rot_ref,
                   o_ref):
    x = x_ref[...]
    rot = rot_ref[...].astype(jnp.bfloat16)
    cos = cos_ref[...]
    sin = sin_ref[...]

    def rope(t):
        tr = jnp.dot(t.astype(jnp.bfloat16), rot,
                     preferred_element_type=jnp.float32)
        return t * cos + tr * sin

    q = jnp.dot(x, wq_ref[...].astype(jnp.bfloat16),
                preferred_element_type=jnp.float32)
    k = jnp.dot(x, wk_ref[...].astype(jnp.bfloat16),
                preferred_element_type=jnp.float32)
    v = jnp.dot(x, wv_ref[...].astype(jnp.bfloat16),
                preferred_element_type=jnp.float32)
    q16 = rope(q).astype(jnp.bfloat16)
    k16 = rope(k).astype(jnp.bfloat16)
    v16 = v.astype(jnp.bfloat16)

    for i in range(SQ // QB):
        s = lax.dot_general(
            q16[i * QB:(i + 1) * QB], k16, (((1,), (1,)), ((), ())),
            preferred_element_type=jnp.float32,
        ) * SCALE
        w = jnp.exp(s)
        denom = jnp.sum(w, axis=-1, keepdims=True)
        o_ref[pl.ds(i * QB, QB), :] = jnp.dot(
            w.astype(jnp.bfloat16), v16,
            preferred_element_type=jnp.float32) / denom


def _qkv_attention(x16, Wq, Wk, Wv):
    w_spec = pl.BlockSpec((D, DH), lambda h: (0, h))
    t_spec = pl.BlockSpec((SQ, DH), lambda h: (0, 0))
    return pl.pallas_call(
        _qkv_attn_body,
        grid=(H_LOC,),
        in_specs=[
            pl.BlockSpec((SQ, D), lambda h: (0, 0)),
            w_spec, w_spec, w_spec,
            t_spec, t_spec,
            pl.BlockSpec((DH, DH), lambda h: (0, 0)),
        ],
        out_specs=pl.BlockSpec((SQ, DH), lambda h: (0, h)),
        out_shape=jax.ShapeDtypeStruct((SQ, D), jnp.float32),
    )(x16, Wq, Wk, Wv, _COS, _SIN, _ROT)


_RS_BASE = (0, 4, 6)
_AG_BASE = (7, 8, 10)
_HALF = D // 2
_BIT_ORDER = ((0, 1, 2), (2, 1, 0))


def _rs_chunks(b, stage_i, me):
    s = _BIT_ORDER[b][stage_i]
    mebit = (me >> s) & 1
    pbit = 1 - mebit
    out = []
    for j in range(4 >> stage_i):
        if b == 0:
            base = me & ((1 << s) - 1)
            c_send = (j << (s + 1)) | (pbit << s) | base
            c_keep = (j << (s + 1)) | (mebit << s) | base
        else:
            base = me & (7 ^ ((1 << (s + 1)) - 1))
            c_send = base | (pbit << s) | j
            c_keep = base | (mebit << s) | j
        out.append((c_send, c_keep, _RS_BASE[stage_i] + j))
    return out


def _ag_chunks(b, k, me):
    s = _BIT_ORDER[b][2 - k]
    partner_bit = 1 << s
    out = []
    for j in range(1 << k):
        if b == 0:
            c_send = (me & ((1 << (s + 1)) - 1)) | (j << (s + 1))
            c_recv = ((me ^ partner_bit) & ((1 << (s + 1)) - 1)) | (j << (s + 1))
        else:
            mask = 7 ^ ((1 << k) - 1)
            c_send = (me & mask) | j
            c_recv = ((me ^ partner_bit) & mask) | j
        out.append((c_send, c_recv, _AG_BASE[k] + j, s))
    return out


def _proj_ar_body(ctx_ref, wo_ref, o_ref, comm_ref, p16_ref,
                  send_sems, recv_sems):
    me = lax.axis_index("i")

    barrier = pltpu.get_barrier_semaphore()
    for s in range(3):
        pl.semaphore_signal(
            barrier, inc=1, device_id=(me ^ (1 << s),),
            device_id_type=pl.DeviceIdType.MESH,
        )
    pl.semaphore_wait(barrier, 3)

    def _rs_rdma(b, c_send, slot, partner):
        return pltpu.make_async_remote_copy(
            src_ref=p16_ref.at[b, c_send],
            dst_ref=comm_ref.at[b, slot],
            send_sem=send_sems.at[b, slot],
            recv_sem=recv_sems.at[b, slot],
            device_id=(partner,),
            device_id_type=pl.DeviceIdType.MESH,
        )

    pbit_a = 1 - (me & 1)
    pbit_b = 1 - ((me >> 2) & 1)

    wo = wo_ref[...].astype(jnp.bfloat16)
    for c in range(N_DEV):
        p = jnp.dot(
            ctx_ref[pl.ds(c * CHUNK, CHUNK), :].astype(jnp.bfloat16), wo,
            preferred_element_type=jnp.float32,
        )
        o_ref[c] = p
        for b in range(2):
            p16_ref[b, c] = p[:, b * _HALF:(b + 1) * _HALF].astype(jnp.bfloat16)

        @pl.when((c & 1) == pbit_a)
        def _():
            _rs_rdma(0, c, _RS_BASE[0] + (c >> 1), me ^ 1).start()

        @pl.when((c >> 2) == pbit_b)
        def _():
            _rs_rdma(1, c, _RS_BASE[0] + (c & 3), me ^ 4).start()

    def rs_start(b, stage_i):
        s = _BIT_ORDER[b][stage_i]
        for c_send, _, slot in _rs_chunks(b, stage_i, me):
            _rs_rdma(b, c_send, slot, me ^ (1 << s)).start()

    def rs_fin(b, stage_i):
        s = _BIT_ORDER[b][stage_i]
        for c_send, c_keep, slot in _rs_chunks(b, stage_i, me):
            _rs_rdma(b, c_send, slot, me ^ (1 << s)).wait()
            acc = (o_ref[c_keep, :, pl.ds(b * _HALF, _HALF)]
                   + comm_ref[b, slot].astype(jnp.float32))
            o_ref[c_keep, :, pl.ds(b * _HALF, _HALF)] = acc
            p16_ref[b, c_keep] = acc.astype(jnp.bfloat16)

    def ag_start(b, k):
        for c_send, _, slot, s in _ag_chunks(b, k, me):
            _rs_rdma(b, c_send, slot, me ^ (1 << s)).start()

    def ag_fin(b, k):
        for c_send, c_recv, slot, s in _ag_chunks(b, k, me):
            _rs_rdma(b, c_send, slot, me ^ (1 << s)).wait()
            o_ref[c_recv, :, pl.ds(b * _HALF, _HALF)] = (
                comm_ref[b, slot].astype(jnp.float32))
            if k < 2:
                p16_ref[b, c_recv] = comm_ref[b, slot]

    rs_fin(0, 0); rs_start(0, 1)
    rs_fin(1, 0); rs_start(1, 1)
    rs_fin(0, 1); rs_start(0, 2)
    rs_fin(1, 1); rs_start(1, 2)
    rs_fin(0, 2); ag_start(0, 0)
    rs_fin(1, 2); ag_start(1, 0)
    ag_fin(0, 0); ag_start(0, 1)
    ag_fin(1, 0); ag_start(1, 1)
    ag_fin(0, 1); ag_start(0, 2)
    ag_fin(1, 1); ag_start(1, 2)
    ag_fin(0, 2)
    ag_fin(1, 2)


def _proj_allreduce(ctx, Wo):
    n_sems = 2 * (N_DEV - 1)
    return pl.pallas_call(
        _proj_ar_body,
        in_specs=[
            pl.BlockSpec(memory_space=pltpu.VMEM),
            pl.BlockSpec(memory_space=pltpu.VMEM),
        ],
        out_specs=pl.BlockSpec(memory_space=pltpu.VMEM),
        out_shape=jax.ShapeDtypeStruct((N_DEV, CHUNK, D), jnp.float32),
        scratch_shapes=[
            pltpu.VMEM((2, n_sems, CHUNK, _HALF), jnp.bfloat16),
            pltpu.VMEM((2, N_DEV, CHUNK, _HALF), jnp.bfloat16),
            pltpu.SemaphoreType.DMA((2, n_sems)),
            pltpu.SemaphoreType.DMA((2, n_sems)),
        ],
        compiler_params=pltpu.CompilerParams(collective_id=0),
    )(ctx, Wo)


def kernel(x, Wq, Wk, Wv, Wo):
    x16 = x.reshape(SQ, D).astype(jnp.bfloat16)
    ctx = _qkv_attention(x16, Wq, Wk, Wv)
    out = _proj_allreduce(ctx, Wo)
    return out.reshape(1, SQ, D)
